# Initial kernel scaffold; baseline (speedup 1.0000x reference)
import jax
import jax.numpy as jnp
from jax.experimental import pallas as pl


def kernel(xx):
    raise NotImplementedError("write your pallas kernel here")


# speedup vs baseline: 1.1107x; 1.1107x over previous
import functools

import jax
import jax.numpy as jnp
from jax import lax
from jax.experimental import pallas as pl
from jax.experimental.pallas import tpu as pltpu
from jax.experimental.pallas import tpu_sc as plsc

_NCELL = 8
_FH, _FW = 3, 5
_S = 14
_WIN = _FH * _FW


_W = 224
_SPAN = 2 * _W + _NCELL * _S


def _hist_body(xx_hbm, out_hbm, buf, stage):
    c = lax.axis_index("c")
    s = lax.axis_index("s")
    wid = c * 16 + s
    row0 = _S * (wid // 2)
    col0 = (_NCELL * _S) * (wid % 2)
    base = row0 * _W + col0

    pltpu.sync_copy(xx_hbm.at[pl.ds(base, _SPAN)], buf)

    lane = lax.iota(jnp.int32, 16)
    dy = jnp.where(lane < _FW, 0, jnp.where(lane < 2 * _FW, 1, 2))
    dx = lane - _FW * dy
    off = dy * _W + dx
    valid = lane < _WIN

    out = jnp.zeros((16,), jnp.float32)
    for jl in range(_NCELL):
        vals = plsc.load_gather(buf, [off + _S * jl], mask=valid)
        pred = jnp.logical_and(vals <= 0.0, valid)
        cnt = plsc.all_reduce_population_count(pred).astype(jnp.float32)
        out = jnp.where(lane == 2 * jl, cnt, out)
        out = jnp.where(lane == 2 * jl + 1, float(_WIN) - cnt, out)

    stage[...] = out
    pltpu.sync_copy(stage, out_hbm.at[pl.ds(wid * 16, 16)])


@functools.cache
def _hist_sc():
    return functools.partial(
        pl.kernel,
        out_type=jax.ShapeDtypeStruct((512,), jnp.float32),
        mesh=plsc.VectorSubcoreMesh(core_axis_name="c", subcore_axis_name="s"),
        compiler_params=pltpu.CompilerParams(needs_layout_passes=False),
        scratch_types=[
            pltpu.VMEM((_SPAN,), jnp.float32),
            pltpu.VMEM((16,), jnp.float32),
        ],
    )(_hist_body)


def kernel(xx):
    return _hist_sc()(xx.reshape(-1)).reshape(16, 16, 2)
